# Initial kernel scaffold; baseline (speedup 1.0000x reference)
import jax
import jax.numpy as jnp
from jax.experimental import pallas as pl


def kernel(x, edge_index, edge_attr, batch, W_gat, att_src, att_dst, W_edge, att_edge, b_gat, prelu_a, W_gcn, b_gcn):
    raise NotImplementedError("write your pallas kernel here")


# speedup vs baseline: 61.4080x; 61.4080x over previous
import functools

import jax
import jax.numpy as jnp
from jax import lax
from jax.experimental import pallas as pl
from jax.experimental.pallas import tpu as pltpu
from jax.experimental.pallas import tpu_sc as plsc

NN = 10000
NNP = 10240
EE = 160000
HH = 512
NF = 4
NC = 2
NS = 16
NW = NC * NS
LN = 16
EPT = 5120
EP = EPT * NW
NSUB = 4
ESUB = EPT // NSUB
CHUNKS = ESUB // LN
NB = 2048


def _prep_body(x_ref, eap_ref, wg_ref, asrc_ref, adst_ref, wedge_ref,
               aedge_ref, ad_out, aself_out, consts_out):
    x = x_ref[...]
    wg = wg_ref[...]
    vs = jnp.sum(wg * asrc_ref[...][None, :], axis=1)
    vd = jnp.sum(wg * adst_ref[...][None, :], axis=1)
    c_e = jnp.sum(wedge_ref[...][0, :] * aedge_ref[...])
    a_s = jnp.sum(x * vs[None, :], axis=1)
    a_d = jnp.sum(x * vd[None, :], axis=1)
    eap = eap_ref[...]
    mean_ea = jnp.sum(eap) / EE
    asl = a_s + a_d + c_e * mean_ea
    asl = jnp.where(asl > 0, asl, 0.2 * asl)
    ub = jnp.max(a_s) + jnp.max(a_d) + jnp.max(c_e * eap)
    gub = jnp.maximum(jnp.where(ub > 0, ub, 0.2 * ub), jnp.max(asl))
    ad_out[...] = a_d
    aself_out[...] = asl
    rows = [jnp.full((128,), c_e, jnp.float32),
            jnp.full((128,), gub, jnp.float32),
            jnp.full((128,), vs[0], jnp.float32),
            jnp.full((128,), vs[1], jnp.float32),
            jnp.full((128,), vs[2], jnp.float32),
            jnp.full((128,), vs[3], jnp.float32),
            jnp.zeros((128,), jnp.float32),
            jnp.zeros((128,), jnp.float32)]
    consts_out[...] = jnp.stack(rows)


def _prep(x, eap, w_gat, att_src, att_dst, w_edge, att_edge):
    return pl.pallas_call(
        _prep_body,
        out_shape=(jax.ShapeDtypeStruct((NNP,), jnp.float32),
                   jax.ShapeDtypeStruct((NNP,), jnp.float32),
                   jax.ShapeDtypeStruct((8, 128), jnp.float32)),
    )(x, eap, w_gat, att_src, att_dst, w_edge, att_edge)


def _gat_edges_body(srcp, dstp, eap, ad_hbm, xt_hbm, consts_hbm,
                    denomp, nump, degp,
                    ad_t, x0_t, x1_t, x2_t, x3_t,
                    den_a, n0_a, n1_a, n2_a, n3_a, deg_a,
                    src_b, dst_b, ea_b, consts_v):
    wid = lax.axis_index("c") * NS + lax.axis_index("s")
    base = wid * EPT
    pltpu.sync_copy(ad_hbm, ad_t)
    pltpu.sync_copy(xt_hbm.at[0], x0_t)
    pltpu.sync_copy(xt_hbm.at[1], x1_t)
    pltpu.sync_copy(xt_hbm.at[2], x2_t)
    pltpu.sync_copy(xt_hbm.at[3], x3_t)
    pltpu.sync_copy(consts_hbm, consts_v)
    c_e = consts_v[0, pl.ds(0, LN)]
    gub = consts_v[1, pl.ds(0, LN)]
    vs0 = consts_v[2, pl.ds(0, LN)]
    vs1 = consts_v[3, pl.ds(0, LN)]
    vs2 = consts_v[4, pl.ds(0, LN)]
    vs3 = consts_v[5, pl.ds(0, LN)]

    def zero_body(j, _):
        z = jnp.zeros((LN,), jnp.float32)
        for r in (den_a, n0_a, n1_a, n2_a, n3_a, deg_a):
            r[pl.ds(j * LN, LN)] = z
        return 0
    lax.fori_loop(0, NNP // LN, zero_body, 0)

    for b in range(NSUB):
        off = base + b * ESUB
        pltpu.sync_copy(srcp.at[pl.ds(off, ESUB)], src_b)
        pltpu.sync_copy(dstp.at[pl.ds(off, ESUB)], dst_b)
        pltpu.sync_copy(eap.at[pl.ds(off, ESUB)], ea_b)

        def body(i, _):
            s = src_b[pl.ds(i * LN, LN)]
            d = dst_b[pl.ds(i * LN, LN)]
            e = ea_b[pl.ds(i * LN, LN)]
            x0 = plsc.load_gather(x0_t, [s])
            x1 = plsc.load_gather(x1_t, [s])
            x2 = plsc.load_gather(x2_t, [s])
            x3 = plsc.load_gather(x3_t, [s])
            adv = plsc.load_gather(ad_t, [d])
            al = x0 * vs0 + x1 * vs1 + x2 * vs2 + x3 * vs3 + adv + c_e * e
            al = jnp.where(al > 0, al, 0.2 * al)
            ae = jnp.exp(al - gub)
            eidx = off + i * LN + lax.iota(jnp.int32, LN)
            ae = jnp.where(eidx < EE, ae, 0.0)
            plsc.addupdate_scatter(den_a, [d], ae)
            plsc.addupdate_scatter(n0_a, [d], ae * x0)
            plsc.addupdate_scatter(n1_a, [d], ae * x1)
            plsc.addupdate_scatter(n2_a, [d], ae * x2)
            plsc.addupdate_scatter(n3_a, [d], ae * x3)
            plsc.addupdate_scatter(deg_a, [d], e)
            return 0
        lax.fori_loop(0, CHUNKS, body, 0)

    pltpu.sync_copy(den_a, denomp.at[wid])
    pltpu.sync_copy(n0_a, nump.at[wid, 0])
    pltpu.sync_copy(n1_a, nump.at[wid, 1])
    pltpu.sync_copy(n2_a, nump.at[wid, 2])
    pltpu.sync_copy(n3_a, nump.at[wid, 3])
    pltpu.sync_copy(deg_a, degp.at[wid])


def _gat_edges(srcp, dstp, eap, ad_arr, xt, consts):
    f = functools.partial(
        pl.kernel,
        out_type=(jax.ShapeDtypeStruct((NW, NNP), jnp.float32),
                  jax.ShapeDtypeStruct((NW, NF, NNP), jnp.float32),
                  jax.ShapeDtypeStruct((NW, NNP), jnp.float32)),
        mesh=plsc.VectorSubcoreMesh(core_axis_name="c", subcore_axis_name="s", num_cores=NC, num_subcores=NS),
        compiler_params=pltpu.CompilerParams(needs_layout_passes=False),
        scratch_types=[
            pltpu.VMEM((NNP,), jnp.float32),
            pltpu.VMEM((NNP,), jnp.float32),
            pltpu.VMEM((NNP,), jnp.float32),
            pltpu.VMEM((NNP,), jnp.float32),
            pltpu.VMEM((NNP,), jnp.float32),
            pltpu.VMEM((NNP,), jnp.float32),
            pltpu.VMEM((NNP,), jnp.float32),
            pltpu.VMEM((NNP,), jnp.float32),
            pltpu.VMEM((NNP,), jnp.float32),
            pltpu.VMEM((NNP,), jnp.float32),
            pltpu.VMEM((NNP,), jnp.float32),
            pltpu.VMEM((ESUB,), jnp.int32),
            pltpu.VMEM((ESUB,), jnp.int32),
            pltpu.VMEM((ESUB,), jnp.float32),
            pltpu.VMEM((8, 128), jnp.float32),
        ],
    )(_gat_edges_body)
    return f(srcp, dstp, eap, ad_arr, xt, consts)


def _mid_body(degp_ref, denomp_ref, nump_ref, aself_ref, xt_ref, consts_ref,
              wg_ref, bg_ref, pa_ref, wgcn_ref, g_out, dinv_out, sp_out):
    deg = jnp.sum(degp_ref[...], axis=0) + 1.0
    dinv = lax.rsqrt(deg)
    gub = consts_ref[...][1, 0]
    es = jnp.exp(aself_ref[...] - gub)
    denom = jnp.sum(denomp_ref[...], axis=0) + es
    numt = jnp.sum(nump_ref[...], axis=0) + es[None, :] * xt_ref[...]
    acc4 = numt / (denom + 1e-16)[None, :]
    x1 = lax.dot_general(acc4, wg_ref[...], (((0,), (0,)), ((), ())),
                         preferred_element_type=jnp.float32)
    x1 = x1 + bg_ref[...][None, :]
    pa = pa_ref[...][0, 0]
    x2 = jnp.where(x1 > 0, x1, pa * x1)
    h2 = jnp.sum(x2 * wgcn_ref[...][:, 0][None, :], axis=1)
    g_out[...] = dinv * h2
    dinv_out[...] = dinv
    sp_out[...] = dinv * dinv * h2


def _mid(degp, denomp, nump, aself, xt, consts, w_gat, b_gat, pa, w_gcn):
    grid = NNP // NB
    return pl.pallas_call(
        _mid_body,
        grid=(grid,),
        in_specs=[
            pl.BlockSpec((NW, NB), lambda i: (0, i)),
            pl.BlockSpec((NW, NB), lambda i: (0, i)),
            pl.BlockSpec((NW, NF, NB), lambda i: (0, 0, i)),
            pl.BlockSpec((NB,), lambda i: (i,)),
            pl.BlockSpec((NF, NB), lambda i: (0, i)),
            pl.BlockSpec((8, 128), lambda i: (0, 0)),
            pl.BlockSpec((NF, HH), lambda i: (0, 0)),
            pl.BlockSpec((HH,), lambda i: (0,)),
            pl.BlockSpec((1, 1), lambda i: (0, 0)),
            pl.BlockSpec((HH, 1), lambda i: (0, 0)),
        ],
        out_specs=[
            pl.BlockSpec((NB,), lambda i: (i,)),
            pl.BlockSpec((NB,), lambda i: (i,)),
            pl.BlockSpec((NB,), lambda i: (i,)),
        ],
        out_shape=(jax.ShapeDtypeStruct((NNP,), jnp.float32),
                   jax.ShapeDtypeStruct((NNP,), jnp.float32),
                   jax.ShapeDtypeStruct((NNP,), jnp.float32)),
    )(degp, denomp, nump, aself, xt, consts, w_gat, b_gat, pa, w_gcn)


def _gcn_edges_body(srcp, dstp, eap, g_hbm, accp,
                    g_t, acc_a, src_b, dst_b, ea_b):
    wid = lax.axis_index("c") * NS + lax.axis_index("s")
    base = wid * EPT
    pltpu.sync_copy(g_hbm, g_t)

    def zero_body(j, _):
        acc_a[pl.ds(j * LN, LN)] = jnp.zeros((LN,), jnp.float32)
        return 0
    lax.fori_loop(0, NNP // LN, zero_body, 0)

    for b in range(NSUB):
        off = base + b * ESUB
        pltpu.sync_copy(srcp.at[pl.ds(off, ESUB)], src_b)
        pltpu.sync_copy(dstp.at[pl.ds(off, ESUB)], dst_b)
        pltpu.sync_copy(eap.at[pl.ds(off, ESUB)], ea_b)

        def body(i, _):
            s = src_b[pl.ds(i * LN, LN)]
            d = dst_b[pl.ds(i * LN, LN)]
            e = ea_b[pl.ds(i * LN, LN)]
            gv = plsc.load_gather(g_t, [s])
            plsc.addupdate_scatter(acc_a, [d], e * gv)
            return 0
        lax.fori_loop(0, CHUNKS, body, 0)

    pltpu.sync_copy(acc_a, accp.at[wid])


def _gcn_edges(srcp, dstp, eap, g):
    f = functools.partial(
        pl.kernel,
        out_type=jax.ShapeDtypeStruct((NW, NNP), jnp.float32),
        mesh=plsc.VectorSubcoreMesh(core_axis_name="c", subcore_axis_name="s", num_cores=NC, num_subcores=NS),
        compiler_params=pltpu.CompilerParams(needs_layout_passes=False),
        scratch_types=[
            pltpu.VMEM((NNP,), jnp.float32),
            pltpu.VMEM((NNP,), jnp.float32),
            pltpu.VMEM((ESUB,), jnp.int32),
            pltpu.VMEM((ESUB,), jnp.int32),
            pltpu.VMEM((ESUB,), jnp.float32),
        ],
    )(_gcn_edges_body)
    return f(srcp, dstp, eap, g)


def _final_body(accp_ref, dinv_ref, sp_ref, bgcn_ref, out_ref):
    acc = jnp.sum(accp_ref[...], axis=0)
    out_ref[...] = jax.nn.sigmoid(dinv_ref[...] * acc + sp_ref[...]
                                  + bgcn_ref[...][0, 0])


def _final(accp, dinv, sp, b_gcn):
    return pl.pallas_call(
        _final_body,
        out_shape=jax.ShapeDtypeStruct((NNP,), jnp.float32),
    )(accp, dinv, sp, b_gcn)


def kernel(x, edge_index, edge_attr, batch, W_gat, att_src, att_dst, W_edge,
           att_edge, b_gat, prelu_a, W_gcn, b_gcn):
    pad = EP - EE
    srcp = jnp.concatenate([edge_index[0], jnp.zeros((pad,), jnp.int32)])
    dstp = jnp.concatenate([edge_index[1], jnp.zeros((pad,), jnp.int32)])
    eap = jnp.concatenate([edge_attr, jnp.zeros((pad,), jnp.float32)])
    xp = jnp.concatenate([x, jnp.zeros((NNP - NN, NF), jnp.float32)])
    xt = xp.T
    ad_arr, aself, consts = _prep(xp, eap, W_gat, att_src, att_dst,
                                  W_edge, att_edge)
    denomp, nump, degp = _gat_edges(srcp, dstp, eap, ad_arr, xt, consts)
    g, dinv, sp = _mid(degp, denomp, nump, aself, xt, consts,
                       W_gat, b_gat, prelu_a.reshape(1, 1), W_gcn)
    accp = _gcn_edges(srcp, dstp, eap, g)
    out = _final(accp, dinv, sp, b_gcn.reshape(1, 1))
    return out[:NN].reshape(NN, 1)


# speedup vs baseline: 74.8246x; 1.2185x over previous
import functools

import jax
import jax.numpy as jnp
from jax import lax
from jax.experimental import pallas as pl
from jax.experimental.pallas import tpu as pltpu
from jax.experimental.pallas import tpu_sc as plsc

NN = 10000
NNP = 10240
EE = 160000
HH = 512
NF = 4
NC = 2
NS = 16
NW = NC * NS
LN = 16
EPT = 5120
EP = EPT * NW
NSUB = 4
ESUB = EPT // NSUB
CHUNKS = ESUB // LN
NB = 2048


def _prep_body(x_ref, eap_ref, wg_ref, asrc_ref, adst_ref, wedge_ref,
               aedge_ref, ad_out, aself_out, consts_out):
    x = x_ref[...]
    wg = wg_ref[...]
    vs = jnp.sum(wg * asrc_ref[...][None, :], axis=1)
    vd = jnp.sum(wg * adst_ref[...][None, :], axis=1)
    c_e = jnp.sum(wedge_ref[...][0, :] * aedge_ref[...])
    a_s = jnp.sum(x * vs[None, :], axis=1)
    a_d = jnp.sum(x * vd[None, :], axis=1)
    eap = eap_ref[...]
    mean_ea = jnp.sum(eap) / EE
    asl = a_s + a_d + c_e * mean_ea
    asl = jnp.where(asl > 0, asl, 0.2 * asl)
    ub = jnp.max(a_s) + jnp.max(a_d) + jnp.max(c_e * eap)
    gub = jnp.maximum(jnp.where(ub > 0, ub, 0.2 * ub), jnp.max(asl))
    ad_out[...] = a_d
    aself_out[...] = asl
    rows = [jnp.full((128,), c_e, jnp.float32),
            jnp.full((128,), gub, jnp.float32),
            jnp.full((128,), vs[0], jnp.float32),
            jnp.full((128,), vs[1], jnp.float32),
            jnp.full((128,), vs[2], jnp.float32),
            jnp.full((128,), vs[3], jnp.float32),
            jnp.zeros((128,), jnp.float32),
            jnp.zeros((128,), jnp.float32)]
    consts_out[...] = jnp.stack(rows)


def _prep(x, eap, w_gat, att_src, att_dst, w_edge, att_edge):
    return pl.pallas_call(
        _prep_body,
        out_shape=(jax.ShapeDtypeStruct((NNP,), jnp.float32),
                   jax.ShapeDtypeStruct((NNP,), jnp.float32),
                   jax.ShapeDtypeStruct((8, 128), jnp.float32)),
    )(x, eap, w_gat, att_src, att_dst, w_edge, att_edge)


def _gat_edges_body(srcp, dstp, eap, ad_hbm, xt_hbm, consts_hbm,
                    denomp, nump, degp,
                    ad_t, x0_t, x1_t, x2_t, x3_t,
                    den_a, n0_a, n1_a, n2_a, n3_a, deg_a,
                    src_b0, dst_b0, ea_b0, src_b1, dst_b1, ea_b1,
                    consts_v, sem_t, sem_e0, sem_e1):
    wid = lax.axis_index("c") * NS + lax.axis_index("s")
    base = wid * EPT
    bufs = ((src_b0, dst_b0, ea_b0), (src_b1, dst_b1, ea_b1))
    sems = (sem_e0, sem_e1)

    def issue(b, slot):
        off = base + b * ESUB
        return (pltpu.async_copy(srcp.at[pl.ds(off, ESUB)], bufs[slot][0],
                                 sems[slot]),
                pltpu.async_copy(dstp.at[pl.ds(off, ESUB)], bufs[slot][1],
                                 sems[slot]),
                pltpu.async_copy(eap.at[pl.ds(off, ESUB)], bufs[slot][2],
                                 sems[slot]))

    tcs = (pltpu.async_copy(ad_hbm, ad_t, sem_t),
           pltpu.async_copy(xt_hbm.at[0], x0_t, sem_t),
           pltpu.async_copy(xt_hbm.at[1], x1_t, sem_t),
           pltpu.async_copy(xt_hbm.at[2], x2_t, sem_t),
           pltpu.async_copy(xt_hbm.at[3], x3_t, sem_t),
           pltpu.async_copy(consts_hbm, consts_v, sem_t))
    pend = issue(0, 0)

    @plsc.parallel_loop(0, NNP // LN, unroll=8)
    def _(j):
        z = jnp.zeros((LN,), jnp.float32)
        for r in (den_a, n0_a, n1_a, n2_a, n3_a, deg_a):
            r[pl.ds(j * LN, LN)] = z

    for t in tcs:
        t.wait()
    c_e = consts_v[0, pl.ds(0, LN)]
    gub = consts_v[1, pl.ds(0, LN)]
    vs0 = consts_v[2, pl.ds(0, LN)]
    vs1 = consts_v[3, pl.ds(0, LN)]
    vs2 = consts_v[4, pl.ds(0, LN)]
    vs3 = consts_v[5, pl.ds(0, LN)]

    for b in range(NSUB):
        nxt = issue(b + 1, (b + 1) % 2) if b + 1 < NSUB else None
        for dsc in pend:
            dsc.wait()
        sb, db, eb = bufs[b % 2]
        off = base + b * ESUB

        @plsc.parallel_loop(0, CHUNKS, unroll=4)
        def _(i):
            s = sb[pl.ds(i * LN, LN)]
            d = db[pl.ds(i * LN, LN)]
            e = eb[pl.ds(i * LN, LN)]
            x0 = plsc.load_gather(x0_t, [s])
            x1 = plsc.load_gather(x1_t, [s])
            x2 = plsc.load_gather(x2_t, [s])
            x3 = plsc.load_gather(x3_t, [s])
            adv = plsc.load_gather(ad_t, [d])
            al = x0 * vs0 + x1 * vs1 + x2 * vs2 + x3 * vs3 + adv + c_e * e
            al = jnp.where(al > 0, al, 0.2 * al)
            ae = jnp.exp(al - gub)
            eidx = off + i * LN + lax.iota(jnp.int32, LN)
            ae = jnp.where(eidx < EE, ae, 0.0)
            plsc.addupdate_scatter(den_a, [d], ae)
            plsc.addupdate_scatter(n0_a, [d], ae * x0)
            plsc.addupdate_scatter(n1_a, [d], ae * x1)
            plsc.addupdate_scatter(n2_a, [d], ae * x2)
            plsc.addupdate_scatter(n3_a, [d], ae * x3)
            plsc.addupdate_scatter(deg_a, [d], e)
        pend = nxt

    pltpu.sync_copy(den_a, denomp.at[wid])
    pltpu.sync_copy(n0_a, nump.at[wid, 0])
    pltpu.sync_copy(n1_a, nump.at[wid, 1])
    pltpu.sync_copy(n2_a, nump.at[wid, 2])
    pltpu.sync_copy(n3_a, nump.at[wid, 3])
    pltpu.sync_copy(deg_a, degp.at[wid])


def _gat_edges(srcp, dstp, eap, ad_arr, xt, consts):
    f = functools.partial(
        pl.kernel,
        out_type=(jax.ShapeDtypeStruct((NW, NNP), jnp.float32),
                  jax.ShapeDtypeStruct((NW, NF, NNP), jnp.float32),
                  jax.ShapeDtypeStruct((NW, NNP), jnp.float32)),
        mesh=plsc.VectorSubcoreMesh(core_axis_name="c", subcore_axis_name="s", num_cores=NC, num_subcores=NS),
        compiler_params=pltpu.CompilerParams(needs_layout_passes=False),
        scratch_types=[
            pltpu.VMEM((NNP,), jnp.float32),
            pltpu.VMEM((NNP,), jnp.float32),
            pltpu.VMEM((NNP,), jnp.float32),
            pltpu.VMEM((NNP,), jnp.float32),
            pltpu.VMEM((NNP,), jnp.float32),
            pltpu.VMEM((NNP,), jnp.float32),
            pltpu.VMEM((NNP,), jnp.float32),
            pltpu.VMEM((NNP,), jnp.float32),
            pltpu.VMEM((NNP,), jnp.float32),
            pltpu.VMEM((NNP,), jnp.float32),
            pltpu.VMEM((NNP,), jnp.float32),
            pltpu.VMEM((ESUB,), jnp.int32),
            pltpu.VMEM((ESUB,), jnp.int32),
            pltpu.VMEM((ESUB,), jnp.float32),
            pltpu.VMEM((ESUB,), jnp.int32),
            pltpu.VMEM((ESUB,), jnp.int32),
            pltpu.VMEM((ESUB,), jnp.float32),
            pltpu.VMEM((8, 128), jnp.float32),
            pltpu.SemaphoreType.DMA,
            pltpu.SemaphoreType.DMA,
            pltpu.SemaphoreType.DMA,
        ],
    )(_gat_edges_body)
    return f(srcp, dstp, eap, ad_arr, xt, consts)


def _mid_body(degp_ref, denomp_ref, nump_ref, aself_ref, xt_ref, consts_ref,
              wg_ref, bg_ref, pa_ref, wgcn_ref, g_out, dinv_out, sp_out):
    deg = jnp.sum(degp_ref[...], axis=0) + 1.0
    dinv = lax.rsqrt(deg)
    gub = consts_ref[...][1, 0]
    es = jnp.exp(aself_ref[...] - gub)
    denom = jnp.sum(denomp_ref[...], axis=0) + es
    numt = jnp.sum(nump_ref[...], axis=0) + es[None, :] * xt_ref[...]
    acc4 = numt / (denom + 1e-16)[None, :]
    x1 = lax.dot_general(acc4, wg_ref[...], (((0,), (0,)), ((), ())),
                         preferred_element_type=jnp.float32)
    x1 = x1 + bg_ref[...][None, :]
    pa = pa_ref[...][0, 0]
    x2 = jnp.where(x1 > 0, x1, pa * x1)
    h2 = jnp.sum(x2 * wgcn_ref[...][:, 0][None, :], axis=1)
    g_out[...] = dinv * h2
    dinv_out[...] = dinv
    sp_out[...] = dinv * dinv * h2


def _mid(degp, denomp, nump, aself, xt, consts, w_gat, b_gat, pa, w_gcn):
    grid = NNP // NB
    return pl.pallas_call(
        _mid_body,
        grid=(grid,),
        in_specs=[
            pl.BlockSpec((NW, NB), lambda i: (0, i)),
            pl.BlockSpec((NW, NB), lambda i: (0, i)),
            pl.BlockSpec((NW, NF, NB), lambda i: (0, 0, i)),
            pl.BlockSpec((NB,), lambda i: (i,)),
            pl.BlockSpec((NF, NB), lambda i: (0, i)),
            pl.BlockSpec((8, 128), lambda i: (0, 0)),
            pl.BlockSpec((NF, HH), lambda i: (0, 0)),
            pl.BlockSpec((HH,), lambda i: (0,)),
            pl.BlockSpec((1, 1), lambda i: (0, 0)),
            pl.BlockSpec((HH, 1), lambda i: (0, 0)),
        ],
        out_specs=[
            pl.BlockSpec((NB,), lambda i: (i,)),
            pl.BlockSpec((NB,), lambda i: (i,)),
            pl.BlockSpec((NB,), lambda i: (i,)),
        ],
        out_shape=(jax.ShapeDtypeStruct((NNP,), jnp.float32),
                   jax.ShapeDtypeStruct((NNP,), jnp.float32),
                   jax.ShapeDtypeStruct((NNP,), jnp.float32)),
    )(degp, denomp, nump, aself, xt, consts, w_gat, b_gat, pa, w_gcn)


def _gcn_edges_body(srcp, dstp, eap, g_hbm, accp,
                    g_t, acc_a, src_b0, dst_b0, ea_b0, src_b1, dst_b1, ea_b1,
                    sem_t, sem_e0, sem_e1):
    wid = lax.axis_index("c") * NS + lax.axis_index("s")
    base = wid * EPT
    bufs = ((src_b0, dst_b0, ea_b0), (src_b1, dst_b1, ea_b1))
    sems = (sem_e0, sem_e1)

    def issue(b, slot):
        off = base + b * ESUB
        return (pltpu.async_copy(srcp.at[pl.ds(off, ESUB)], bufs[slot][0],
                                 sems[slot]),
                pltpu.async_copy(dstp.at[pl.ds(off, ESUB)], bufs[slot][1],
                                 sems[slot]),
                pltpu.async_copy(eap.at[pl.ds(off, ESUB)], bufs[slot][2],
                                 sems[slot]))

    tc = pltpu.async_copy(g_hbm, g_t, sem_t)
    pend = issue(0, 0)

    @plsc.parallel_loop(0, NNP // LN, unroll=8)
    def _(j):
        acc_a[pl.ds(j * LN, LN)] = jnp.zeros((LN,), jnp.float32)

    tc.wait()

    for b in range(NSUB):
        nxt = issue(b + 1, (b + 1) % 2) if b + 1 < NSUB else None
        for dsc in pend:
            dsc.wait()
        sb, db, eb = bufs[b % 2]

        @plsc.parallel_loop(0, CHUNKS, unroll=8)
        def _(i):
            s = sb[pl.ds(i * LN, LN)]
            d = db[pl.ds(i * LN, LN)]
            e = eb[pl.ds(i * LN, LN)]
            gv = plsc.load_gather(g_t, [s])
            plsc.addupdate_scatter(acc_a, [d], e * gv)
        pend = nxt

    pltpu.sync_copy(acc_a, accp.at[wid])


def _gcn_edges(srcp, dstp, eap, g):
    f = functools.partial(
        pl.kernel,
        out_type=jax.ShapeDtypeStruct((NW, NNP), jnp.float32),
        mesh=plsc.VectorSubcoreMesh(core_axis_name="c", subcore_axis_name="s", num_cores=NC, num_subcores=NS),
        compiler_params=pltpu.CompilerParams(needs_layout_passes=False),
        scratch_types=[
            pltpu.VMEM((NNP,), jnp.float32),
            pltpu.VMEM((NNP,), jnp.float32),
            pltpu.VMEM((ESUB,), jnp.int32),
            pltpu.VMEM((ESUB,), jnp.int32),
            pltpu.VMEM((ESUB,), jnp.float32),
            pltpu.VMEM((ESUB,), jnp.int32),
            pltpu.VMEM((ESUB,), jnp.int32),
            pltpu.VMEM((ESUB,), jnp.float32),
            pltpu.SemaphoreType.DMA,
            pltpu.SemaphoreType.DMA,
            pltpu.SemaphoreType.DMA,
        ],
    )(_gcn_edges_body)
    return f(srcp, dstp, eap, g)


def _final_body(accp_ref, dinv_ref, sp_ref, bgcn_ref, out_ref):
    acc = jnp.sum(accp_ref[...], axis=0)
    out_ref[...] = jax.nn.sigmoid(dinv_ref[...] * acc + sp_ref[...]
                                  + bgcn_ref[...][0, 0])


def _final(accp, dinv, sp, b_gcn):
    return pl.pallas_call(
        _final_body,
        out_shape=jax.ShapeDtypeStruct((NNP,), jnp.float32),
    )(accp, dinv, sp, b_gcn)


def kernel(x, edge_index, edge_attr, batch, W_gat, att_src, att_dst, W_edge,
           att_edge, b_gat, prelu_a, W_gcn, b_gcn):
    pad = EP - EE
    srcp = jnp.concatenate([edge_index[0], jnp.zeros((pad,), jnp.int32)])
    dstp = jnp.concatenate([edge_index[1], jnp.zeros((pad,), jnp.int32)])
    eap = jnp.concatenate([edge_attr, jnp.zeros((pad,), jnp.float32)])
    xp = jnp.concatenate([x, jnp.zeros((NNP - NN, NF), jnp.float32)])
    xt = xp.T
    ad_arr, aself, consts = _prep(xp, eap, W_gat, att_src, att_dst,
                                  W_edge, att_edge)
    denomp, nump, degp = _gat_edges(srcp, dstp, eap, ad_arr, xt, consts)
    g, dinv, sp = _mid(degp, denomp, nump, aself, xt, consts,
                       W_gat, b_gat, prelu_a.reshape(1, 1), W_gcn)
    accp = _gcn_edges(srcp, dstp, eap, g)
    out = _final(accp, dinv, sp, b_gcn.reshape(1, 1))
    return out[:NN].reshape(NN, 1)


# speedup vs baseline: 126.5559x; 1.6914x over previous
import functools

import jax
import jax.numpy as jnp
from jax import lax
from jax.experimental import pallas as pl
from jax.experimental.pallas import tpu as pltpu
from jax.experimental.pallas import tpu_sc as plsc

NN = 10000
NNP = 10240
EE = 160000
HH = 512
NF = 4
NC = 2
NS = 16
NW = NC * NS
LN = 16
EPT = EE // NW
EBUF = 5008
CHF = EPT // LN
TAIL = EPT - CHF * LN
NB = 2048


def _prep_body(x_ref, ea2_ref, wg_ref, asrc_ref, adst_ref, wedge_ref,
               aedge_ref, xt_out, ad_out, aself_out, consts_out):
    x = x_ref[...]
    wg = wg_ref[...]
    vs = jnp.sum(wg * asrc_ref[...][None, :], axis=1)
    vd = jnp.sum(wg * adst_ref[...][None, :], axis=1)
    c_e = jnp.sum(wedge_ref[...][0, :] * aedge_ref[...])
    xt = lax.dot_general(jnp.eye(NF, dtype=jnp.float32), x,
                         (((1,), (1,)), ((), ())),
                         preferred_element_type=jnp.float32)
    xtp = jnp.concatenate(
        [xt, jnp.zeros((NF, NNP - NN), jnp.float32)], axis=1)
    a_s = (vs[0] * xtp[0] + vs[1] * xtp[1]
           + vs[2] * xtp[2] + vs[3] * xtp[3])
    a_d = (vd[0] * xtp[0] + vd[1] * xtp[1]
           + vd[2] * xtp[2] + vd[3] * xtp[3])
    ea2 = ea2_ref[...]
    mean_ea = jnp.sum(ea2) / EE
    asl = a_s + a_d + c_e * mean_ea
    asl = jnp.where(asl > 0, asl, 0.2 * asl)
    ub = jnp.max(a_s) + jnp.max(a_d) + jnp.max(c_e * ea2)
    gub = jnp.maximum(jnp.where(ub > 0, ub, 0.2 * ub), jnp.max(asl))
    xt_out[...] = xtp
    ad_out[...] = a_d
    aself_out[...] = asl
    rows = [jnp.full((128,), c_e, jnp.float32),
            jnp.full((128,), gub, jnp.float32),
            jnp.full((128,), vs[0], jnp.float32),
            jnp.full((128,), vs[1], jnp.float32),
            jnp.full((128,), vs[2], jnp.float32),
            jnp.full((128,), vs[3], jnp.float32),
            jnp.zeros((128,), jnp.float32),
            jnp.zeros((128,), jnp.float32)]
    consts_out[...] = jnp.stack(rows)


def _prep(x, ea2, w_gat, att_src, att_dst, w_edge, att_edge):
    return pl.pallas_call(
        _prep_body,
        out_shape=(jax.ShapeDtypeStruct((NF, NNP), jnp.float32),
                   jax.ShapeDtypeStruct((NNP,), jnp.float32),
                   jax.ShapeDtypeStruct((NNP,), jnp.float32),
                   jax.ShapeDtypeStruct((8, 128), jnp.float32)),
    )(x, ea2, w_gat, att_src, att_dst, w_edge, att_edge)


def _gat_edges_body(src_hbm, dst_hbm, ea_hbm, ad_hbm, xt_hbm, consts_hbm,
                    denomp, nump, degp,
                    ad_t, x0_t, x1_t, x2_t, x3_t,
                    den_a, n0_a, n1_a, n2_a, n3_a, deg_a,
                    src_b, dst_b, ea_b, consts_v, sem_t, sem_e):
    wid = lax.axis_index("c") * NS + lax.axis_index("s")
    base = wid * EPT

    ecs = (pltpu.async_copy(src_hbm.at[pl.ds(base, EPT)],
                            src_b.at[pl.ds(0, EPT)], sem_e),
           pltpu.async_copy(dst_hbm.at[pl.ds(base, EPT)],
                            dst_b.at[pl.ds(0, EPT)], sem_e),
           pltpu.async_copy(ea_hbm.at[pl.ds(base, EPT)],
                            ea_b.at[pl.ds(0, EPT)], sem_e))
    tcs = (pltpu.async_copy(ad_hbm, ad_t, sem_t),
           pltpu.async_copy(xt_hbm.at[0], x0_t, sem_t),
           pltpu.async_copy(xt_hbm.at[1], x1_t, sem_t),
           pltpu.async_copy(xt_hbm.at[2], x2_t, sem_t),
           pltpu.async_copy(xt_hbm.at[3], x3_t, sem_t),
           pltpu.async_copy(consts_hbm, consts_v, sem_t))

    @plsc.parallel_loop(0, NNP // LN, unroll=8)
    def _(j):
        z = jnp.zeros((LN,), jnp.float32)
        for r in (den_a, n0_a, n1_a, n2_a, n3_a, deg_a):
            r[pl.ds(j * LN, LN)] = z

    for t in tcs + ecs:
        t.wait()
    c_e = consts_v[0, pl.ds(0, LN)]
    gub = consts_v[1, pl.ds(0, LN)]
    vs0 = consts_v[2, pl.ds(0, LN)]
    vs1 = consts_v[3, pl.ds(0, LN)]
    vs2 = consts_v[4, pl.ds(0, LN)]
    vs3 = consts_v[5, pl.ds(0, LN)]

    def chunk(i, mask):
        s = src_b[pl.ds(i * LN, LN)]
        d = dst_b[pl.ds(i * LN, LN)]
        e = ea_b[pl.ds(i * LN, LN)]
        x0 = plsc.load_gather(x0_t, [s], mask=mask)
        x1 = plsc.load_gather(x1_t, [s], mask=mask)
        x2 = plsc.load_gather(x2_t, [s], mask=mask)
        x3 = plsc.load_gather(x3_t, [s], mask=mask)
        adv = plsc.load_gather(ad_t, [d], mask=mask)
        al = x0 * vs0 + x1 * vs1 + x2 * vs2 + x3 * vs3 + adv + c_e * e
        al = jnp.where(al > 0, al, 0.2 * al)
        ae = jnp.exp(al - gub)
        plsc.addupdate_scatter(den_a, [d], ae, mask=mask)
        plsc.addupdate_scatter(n0_a, [d], ae * x0, mask=mask)
        plsc.addupdate_scatter(n1_a, [d], ae * x1, mask=mask)
        plsc.addupdate_scatter(n2_a, [d], ae * x2, mask=mask)
        plsc.addupdate_scatter(n3_a, [d], ae * x3, mask=mask)
        plsc.addupdate_scatter(deg_a, [d], e, mask=mask)

    @plsc.parallel_loop(0, CHF, unroll=4)
    def _(i):
        chunk(i, None)

    chunk(CHF, lax.iota(jnp.int32, LN) < TAIL)

    pltpu.sync_copy(den_a, denomp.at[wid])
    pltpu.sync_copy(n0_a, nump.at[wid, 0])
    pltpu.sync_copy(n1_a, nump.at[wid, 1])
    pltpu.sync_copy(n2_a, nump.at[wid, 2])
    pltpu.sync_copy(n3_a, nump.at[wid, 3])
    pltpu.sync_copy(deg_a, degp.at[wid])


def _gat_edges(srcv, dstv, ea, ad_arr, xt, consts):
    f = functools.partial(
        pl.kernel,
        out_type=(jax.ShapeDtypeStruct((NW, NNP), jnp.float32),
                  jax.ShapeDtypeStruct((NW, NF, NNP), jnp.float32),
                  jax.ShapeDtypeStruct((NW, NNP), jnp.float32)),
        mesh=plsc.VectorSubcoreMesh(core_axis_name="c", subcore_axis_name="s",
                                    num_cores=NC, num_subcores=NS),
        compiler_params=pltpu.CompilerParams(needs_layout_passes=False),
        scratch_types=[
            pltpu.VMEM((NNP,), jnp.float32),
            pltpu.VMEM((NNP,), jnp.float32),
            pltpu.VMEM((NNP,), jnp.float32),
            pltpu.VMEM((NNP,), jnp.float32),
            pltpu.VMEM((NNP,), jnp.float32),
            pltpu.VMEM((NNP,), jnp.float32),
            pltpu.VMEM((NNP,), jnp.float32),
            pltpu.VMEM((NNP,), jnp.float32),
            pltpu.VMEM((NNP,), jnp.float32),
            pltpu.VMEM((NNP,), jnp.float32),
            pltpu.VMEM((NNP,), jnp.float32),
            pltpu.VMEM((EBUF,), jnp.int32),
            pltpu.VMEM((EBUF,), jnp.int32),
            pltpu.VMEM((EBUF,), jnp.float32),
            pltpu.VMEM((8, 128), jnp.float32),
            pltpu.SemaphoreType.DMA,
            pltpu.SemaphoreType.DMA,
        ],
    )(_gat_edges_body)
    return f(srcv, dstv, ea, ad_arr, xt, consts)


def _mid_body(degp_ref, denomp_ref, nump_ref, aself_ref, xt_ref, consts_ref,
              wg_ref, bg_ref, pa_ref, wgcn_ref, g_out, dinv_out, sp_out):
    deg = jnp.sum(degp_ref[...], axis=0) + 1.0
    dinv = lax.rsqrt(deg)
    gub = consts_ref[...][1, 0]
    es = jnp.exp(aself_ref[...] - gub)
    denom = jnp.sum(denomp_ref[...], axis=0) + es
    numt = jnp.sum(nump_ref[...], axis=0) + es[None, :] * xt_ref[...]
    acc4 = numt / (denom + 1e-16)[None, :]
    x1 = lax.dot_general(acc4, wg_ref[...], (((0,), (0,)), ((), ())),
                         preferred_element_type=jnp.float32)
    x1 = x1 + bg_ref[...][None, :]
    pa = pa_ref[...][0, 0]
    x2 = jnp.where(x1 > 0, x1, pa * x1)
    h2 = jnp.sum(x2 * wgcn_ref[...][:, 0][None, :], axis=1)
    g_out[...] = dinv * h2
    dinv_out[...] = dinv
    sp_out[...] = dinv * dinv * h2


def _mid(degp, denomp, nump, aself, xt, consts, w_gat, b_gat, pa, w_gcn):
    grid = NNP // NB
    return pl.pallas_call(
        _mid_body,
        grid=(grid,),
        in_specs=[
            pl.BlockSpec((NW, NB), lambda i: (0, i)),
            pl.BlockSpec((NW, NB), lambda i: (0, i)),
            pl.BlockSpec((NW, NF, NB), lambda i: (0, 0, i)),
            pl.BlockSpec((NB,), lambda i: (i,)),
            pl.BlockSpec((NF, NB), lambda i: (0, i)),
            pl.BlockSpec((8, 128), lambda i: (0, 0)),
            pl.BlockSpec((NF, HH), lambda i: (0, 0)),
            pl.BlockSpec((HH,), lambda i: (0,)),
            pl.BlockSpec((1, 1), lambda i: (0, 0)),
            pl.BlockSpec((HH, 1), lambda i: (0, 0)),
        ],
        out_specs=[
            pl.BlockSpec((NB,), lambda i: (i,)),
            pl.BlockSpec((NB,), lambda i: (i,)),
            pl.BlockSpec((NB,), lambda i: (i,)),
        ],
        out_shape=(jax.ShapeDtypeStruct((NNP,), jnp.float32),
                   jax.ShapeDtypeStruct((NNP,), jnp.float32),
                   jax.ShapeDtypeStruct((NNP,), jnp.float32)),
    )(degp, denomp, nump, aself, xt, consts, w_gat, b_gat, pa, w_gcn)


def _gcn_edges_body(src_hbm, dst_hbm, ea_hbm, g_hbm, accp,
                    g_t, acc_a, src_b, dst_b, ea_b, sem_t, sem_e):
    wid = lax.axis_index("c") * NS + lax.axis_index("s")
    base = wid * EPT

    ecs = (pltpu.async_copy(src_hbm.at[pl.ds(base, EPT)],
                            src_b.at[pl.ds(0, EPT)], sem_e),
           pltpu.async_copy(dst_hbm.at[pl.ds(base, EPT)],
                            dst_b.at[pl.ds(0, EPT)], sem_e),
           pltpu.async_copy(ea_hbm.at[pl.ds(base, EPT)],
                            ea_b.at[pl.ds(0, EPT)], sem_e))
    tc = pltpu.async_copy(g_hbm, g_t, sem_t)

    @plsc.parallel_loop(0, NNP // LN, unroll=8)
    def _(j):
        acc_a[pl.ds(j * LN, LN)] = jnp.zeros((LN,), jnp.float32)

    tc.wait()
    for dsc in ecs:
        dsc.wait()

    def chunk(i, mask):
        s = src_b[pl.ds(i * LN, LN)]
        d = dst_b[pl.ds(i * LN, LN)]
        e = ea_b[pl.ds(i * LN, LN)]
        gv = plsc.load_gather(g_t, [s], mask=mask)
        plsc.addupdate_scatter(acc_a, [d], e * gv, mask=mask)

    @plsc.parallel_loop(0, CHF, unroll=8)
    def _(i):
        chunk(i, None)

    chunk(CHF, lax.iota(jnp.int32, LN) < TAIL)

    pltpu.sync_copy(acc_a, accp.at[wid])


def _gcn_edges(srcv, dstv, ea, g):
    f = functools.partial(
        pl.kernel,
        out_type=jax.ShapeDtypeStruct((NW, NNP), jnp.float32),
        mesh=plsc.VectorSubcoreMesh(core_axis_name="c", subcore_axis_name="s",
                                    num_cores=NC, num_subcores=NS),
        compiler_params=pltpu.CompilerParams(needs_layout_passes=False),
        scratch_types=[
            pltpu.VMEM((NNP,), jnp.float32),
            pltpu.VMEM((NNP,), jnp.float32),
            pltpu.VMEM((EBUF,), jnp.int32),
            pltpu.VMEM((EBUF,), jnp.int32),
            pltpu.VMEM((EBUF,), jnp.float32),
            pltpu.SemaphoreType.DMA,
            pltpu.SemaphoreType.DMA,
        ],
    )(_gcn_edges_body)
    return f(srcv, dstv, ea, g)


def _final_body(accp_ref, dinv_ref, sp_ref, bgcn_ref, out_ref):
    acc = jnp.sum(accp_ref[...], axis=0)
    out_ref[...] = jax.nn.sigmoid(dinv_ref[...] * acc + sp_ref[...]
                                  + bgcn_ref[...][0, 0])


def _final(accp, dinv, sp, b_gcn):
    return pl.pallas_call(
        _final_body,
        out_shape=jax.ShapeDtypeStruct((NNP,), jnp.float32),
    )(accp, dinv, sp, b_gcn)


def kernel(x, edge_index, edge_attr, batch, W_gat, att_src, att_dst, W_edge,
           att_edge, b_gat, prelu_a, W_gcn, b_gcn):
    ea2 = edge_attr.reshape(EE // 128, 128)
    srcv = edge_index[0]
    dstv = edge_index[1]
    xt, ad_arr, aself, consts = _prep(x, ea2, W_gat, att_src, att_dst,
                                      W_edge, att_edge)
    denomp, nump, degp = _gat_edges(srcv, dstv, edge_attr, ad_arr, xt, consts)
    g, dinv, sp = _mid(degp, denomp, nump, aself, xt, consts,
                       W_gat, b_gat, prelu_a.reshape(1, 1), W_gcn)
    accp = _gcn_edges(srcv, dstv, edge_attr, g)
    out = _final(accp, dinv, sp, b_gcn.reshape(1, 1))
    return out[:NN].reshape(NN, 1)


# speedup vs baseline: 136.3202x; 1.0772x over previous
import functools

import jax
import jax.numpy as jnp
from jax import lax
from jax.experimental import pallas as pl
from jax.experimental.pallas import tpu as pltpu
from jax.experimental.pallas import tpu_sc as plsc

NN = 10000
NNP = 10240
EE = 160000
HH = 512
NF = 4
NC = 2
NS = 16
NW = NC * NS
LN = 16
EPT = EE // NW
EBUF = 5008
CHF = EPT // LN
TAIL = EPT - CHF * LN
NB = 2048


def _prep_body(x_ref, ea2_ref, wg_ref, asrc_ref, adst_ref, wedge_ref,
               aedge_ref, xt_out, ad_out, aself_out, consts_out):
    x = x_ref[...]
    wg = wg_ref[...]
    vs = jnp.sum(wg * asrc_ref[...][None, :], axis=1)
    vd = jnp.sum(wg * adst_ref[...][None, :], axis=1)
    c_e = jnp.sum(wedge_ref[...][0, :] * aedge_ref[...])
    xt = lax.dot_general(jnp.eye(NF, dtype=jnp.float32), x,
                         (((1,), (1,)), ((), ())),
                         preferred_element_type=jnp.float32)
    xtp = jnp.concatenate(
        [xt, jnp.zeros((NF, NNP - NN), jnp.float32)], axis=1)
    a_s = (vs[0] * xtp[0] + vs[1] * xtp[1]
           + vs[2] * xtp[2] + vs[3] * xtp[3])
    a_d = (vd[0] * xtp[0] + vd[1] * xtp[1]
           + vd[2] * xtp[2] + vd[3] * xtp[3])
    ea2 = ea2_ref[...]
    mean_ea = jnp.sum(ea2) / EE
    asl = a_s + a_d + c_e * mean_ea
    asl = jnp.where(asl > 0, asl, 0.2 * asl)
    ub = jnp.max(a_s) + jnp.max(a_d) + jnp.max(c_e * ea2)
    gub = jnp.maximum(jnp.where(ub > 0, ub, 0.2 * ub), jnp.max(asl))
    xt_out[...] = xtp
    ad_out[...] = a_d
    aself_out[...] = asl
    rows = [jnp.full((128,), c_e, jnp.float32),
            jnp.full((128,), gub, jnp.float32),
            jnp.full((128,), vs[0], jnp.float32),
            jnp.full((128,), vs[1], jnp.float32),
            jnp.full((128,), vs[2], jnp.float32),
            jnp.full((128,), vs[3], jnp.float32),
            jnp.zeros((128,), jnp.float32),
            jnp.zeros((128,), jnp.float32)]
    consts_out[...] = jnp.stack(rows)


def _prep(x, ea2, w_gat, att_src, att_dst, w_edge, att_edge):
    return pl.pallas_call(
        _prep_body,
        out_shape=(jax.ShapeDtypeStruct((NF, NNP), jnp.float32),
                   jax.ShapeDtypeStruct((NNP,), jnp.float32),
                   jax.ShapeDtypeStruct((NNP,), jnp.float32),
                   jax.ShapeDtypeStruct((8, 128), jnp.float32)),
    )(x, ea2, w_gat, att_src, att_dst, w_edge, att_edge)


def _gat_edges_body(sd_hbm, ea_hbm, ad_hbm, xt_hbm, consts_hbm,
                    denomp, nump, degp,
                    ad_t, x0_t, x1_t, x2_t, x3_t,
                    den_a, n0_a, n1_a, n2_a, n3_a, deg_a,
                    src_b, dst_b, ea_b, consts_v, sem_t, sem_e):
    wid = lax.axis_index("c") * NS + lax.axis_index("s")
    base = wid * EPT

    ecs = (pltpu.async_copy(sd_hbm.at[pl.ds(base, EPT)],
                            src_b.at[pl.ds(0, EPT)], sem_e),
           pltpu.async_copy(sd_hbm.at[pl.ds(EE + base, EPT)],
                            dst_b.at[pl.ds(0, EPT)], sem_e),
           pltpu.async_copy(ea_hbm.at[pl.ds(base, EPT)],
                            ea_b.at[pl.ds(0, EPT)], sem_e))
    tcs = (pltpu.async_copy(ad_hbm, ad_t, sem_t),
           pltpu.async_copy(xt_hbm.at[0], x0_t, sem_t),
           pltpu.async_copy(xt_hbm.at[1], x1_t, sem_t),
           pltpu.async_copy(xt_hbm.at[2], x2_t, sem_t),
           pltpu.async_copy(xt_hbm.at[3], x3_t, sem_t),
           pltpu.async_copy(consts_hbm, consts_v, sem_t))

    @plsc.parallel_loop(0, NNP // LN, unroll=8)
    def _(j):
        z = jnp.zeros((LN,), jnp.float32)
        for r in (den_a, n0_a, n1_a, n2_a, n3_a, deg_a):
            r[pl.ds(j * LN, LN)] = z

    for t in tcs + ecs:
        t.wait()
    c_e = consts_v[0, pl.ds(0, LN)]
    gub = consts_v[1, pl.ds(0, LN)]
    vs0 = consts_v[2, pl.ds(0, LN)]
    vs1 = consts_v[3, pl.ds(0, LN)]
    vs2 = consts_v[4, pl.ds(0, LN)]
    vs3 = consts_v[5, pl.ds(0, LN)]

    def chunk(i, mask):
        s = src_b[pl.ds(i * LN, LN)]
        d = dst_b[pl.ds(i * LN, LN)]
        e = ea_b[pl.ds(i * LN, LN)]
        x0 = plsc.load_gather(x0_t, [s], mask=mask)
        x1 = plsc.load_gather(x1_t, [s], mask=mask)
        x2 = plsc.load_gather(x2_t, [s], mask=mask)
        x3 = plsc.load_gather(x3_t, [s], mask=mask)
        adv = plsc.load_gather(ad_t, [d], mask=mask)
        al = x0 * vs0 + x1 * vs1 + x2 * vs2 + x3 * vs3 + adv + c_e * e
        al = jnp.where(al > 0, al, 0.2 * al)
        ae = jnp.exp(al - gub)
        plsc.addupdate_scatter(den_a, [d], ae, mask=mask)
        plsc.addupdate_scatter(n0_a, [d], ae * x0, mask=mask)
        plsc.addupdate_scatter(n1_a, [d], ae * x1, mask=mask)
        plsc.addupdate_scatter(n2_a, [d], ae * x2, mask=mask)
        plsc.addupdate_scatter(n3_a, [d], ae * x3, mask=mask)
        plsc.addupdate_scatter(deg_a, [d], e, mask=mask)

    @plsc.parallel_loop(0, CHF, unroll=4)
    def _(i):
        chunk(i, None)

    chunk(CHF, lax.iota(jnp.int32, LN) < TAIL)

    pltpu.sync_copy(den_a, denomp.at[wid])
    pltpu.sync_copy(n0_a, nump.at[wid, 0])
    pltpu.sync_copy(n1_a, nump.at[wid, 1])
    pltpu.sync_copy(n2_a, nump.at[wid, 2])
    pltpu.sync_copy(n3_a, nump.at[wid, 3])
    pltpu.sync_copy(deg_a, degp.at[wid])


def _gat_edges(sd, ea, ad_arr, xt, consts):
    f = functools.partial(
        pl.kernel,
        out_type=(jax.ShapeDtypeStruct((NW, NNP), jnp.float32),
                  jax.ShapeDtypeStruct((NW, NF, NNP), jnp.float32),
                  jax.ShapeDtypeStruct((NW, NNP), jnp.float32)),
        mesh=plsc.VectorSubcoreMesh(core_axis_name="c", subcore_axis_name="s",
                                    num_cores=NC, num_subcores=NS),
        compiler_params=pltpu.CompilerParams(needs_layout_passes=False),
        scratch_types=[
            pltpu.VMEM((NNP,), jnp.float32),
            pltpu.VMEM((NNP,), jnp.float32),
            pltpu.VMEM((NNP,), jnp.float32),
            pltpu.VMEM((NNP,), jnp.float32),
            pltpu.VMEM((NNP,), jnp.float32),
            pltpu.VMEM((NNP,), jnp.float32),
            pltpu.VMEM((NNP,), jnp.float32),
            pltpu.VMEM((NNP,), jnp.float32),
            pltpu.VMEM((NNP,), jnp.float32),
            pltpu.VMEM((NNP,), jnp.float32),
            pltpu.VMEM((NNP,), jnp.float32),
            pltpu.VMEM((EBUF,), jnp.int32),
            pltpu.VMEM((EBUF,), jnp.int32),
            pltpu.VMEM((EBUF,), jnp.float32),
            pltpu.VMEM((8, 128), jnp.float32),
            pltpu.SemaphoreType.DMA,
            pltpu.SemaphoreType.DMA,
        ],
    )(_gat_edges_body)
    return f(sd, ea, ad_arr, xt, consts)


def _mid_body(degp_ref, denomp_ref, nump_ref, aself_ref, xt_ref, consts_ref,
              wg_ref, bg_ref, pa_ref, wgcn_ref, g_out, dinv_out, sp_out):
    deg = jnp.sum(degp_ref[...], axis=0) + 1.0
    dinv = lax.rsqrt(deg)
    gub = consts_ref[...][1, 0]
    es = jnp.exp(aself_ref[...] - gub)
    denom = jnp.sum(denomp_ref[...], axis=0) + es
    numt = jnp.sum(nump_ref[...], axis=0) + es[None, :] * xt_ref[...]
    acc4 = numt / (denom + 1e-16)[None, :]
    x1 = lax.dot_general(acc4, wg_ref[...], (((0,), (0,)), ((), ())),
                         preferred_element_type=jnp.float32)
    x1 = x1 + bg_ref[...][None, :]
    pa = pa_ref[...][0, 0]
    x2 = jnp.where(x1 > 0, x1, pa * x1)
    h2 = jnp.sum(x2 * wgcn_ref[...][:, 0][None, :], axis=1)
    g_out[...] = dinv * h2
    dinv_out[...] = dinv
    sp_out[...] = dinv * dinv * h2


def _mid(degp, denomp, nump, aself, xt, consts, w_gat, b_gat, pa, w_gcn):
    grid = NNP // NB
    return pl.pallas_call(
        _mid_body,
        grid=(grid,),
        in_specs=[
            pl.BlockSpec((NW, NB), lambda i: (0, i)),
            pl.BlockSpec((NW, NB), lambda i: (0, i)),
            pl.BlockSpec((NW, NF, NB), lambda i: (0, 0, i)),
            pl.BlockSpec((NB,), lambda i: (i,)),
            pl.BlockSpec((NF, NB), lambda i: (0, i)),
            pl.BlockSpec((8, 128), lambda i: (0, 0)),
            pl.BlockSpec((NF, HH), lambda i: (0, 0)),
            pl.BlockSpec((HH,), lambda i: (0,)),
            pl.BlockSpec((1, 1), lambda i: (0, 0)),
            pl.BlockSpec((HH, 1), lambda i: (0, 0)),
        ],
        out_specs=[
            pl.BlockSpec((NB,), lambda i: (i,)),
            pl.BlockSpec((NB,), lambda i: (i,)),
            pl.BlockSpec((NB,), lambda i: (i,)),
        ],
        out_shape=(jax.ShapeDtypeStruct((NNP,), jnp.float32),
                   jax.ShapeDtypeStruct((NNP,), jnp.float32),
                   jax.ShapeDtypeStruct((NNP,), jnp.float32)),
    )(degp, denomp, nump, aself, xt, consts, w_gat, b_gat, pa, w_gcn)


def _gcn_edges_body(sd_hbm, ea_hbm, g_hbm, accp,
                    g_t, acc_a, src_b, dst_b, ea_b, sem_t, sem_e):
    wid = lax.axis_index("c") * NS + lax.axis_index("s")
    base = wid * EPT

    ecs = (pltpu.async_copy(sd_hbm.at[pl.ds(base, EPT)],
                            src_b.at[pl.ds(0, EPT)], sem_e),
           pltpu.async_copy(sd_hbm.at[pl.ds(EE + base, EPT)],
                            dst_b.at[pl.ds(0, EPT)], sem_e),
           pltpu.async_copy(ea_hbm.at[pl.ds(base, EPT)],
                            ea_b.at[pl.ds(0, EPT)], sem_e))
    tc = pltpu.async_copy(g_hbm, g_t, sem_t)

    @plsc.parallel_loop(0, NNP // LN, unroll=8)
    def _(j):
        acc_a[pl.ds(j * LN, LN)] = jnp.zeros((LN,), jnp.float32)

    tc.wait()
    for dsc in ecs:
        dsc.wait()

    def chunk(i, mask):
        s = src_b[pl.ds(i * LN, LN)]
        d = dst_b[pl.ds(i * LN, LN)]
        e = ea_b[pl.ds(i * LN, LN)]
        gv = plsc.load_gather(g_t, [s], mask=mask)
        plsc.addupdate_scatter(acc_a, [d], e * gv, mask=mask)

    @plsc.parallel_loop(0, CHF, unroll=8)
    def _(i):
        chunk(i, None)

    chunk(CHF, lax.iota(jnp.int32, LN) < TAIL)

    pltpu.sync_copy(acc_a, accp.at[wid])


def _gcn_edges(sd, ea, g):
    f = functools.partial(
        pl.kernel,
        out_type=jax.ShapeDtypeStruct((NW, NNP), jnp.float32),
        mesh=plsc.VectorSubcoreMesh(core_axis_name="c", subcore_axis_name="s",
                                    num_cores=NC, num_subcores=NS),
        compiler_params=pltpu.CompilerParams(needs_layout_passes=False),
        scratch_types=[
            pltpu.VMEM((NNP,), jnp.float32),
            pltpu.VMEM((NNP,), jnp.float32),
            pltpu.VMEM((EBUF,), jnp.int32),
            pltpu.VMEM((EBUF,), jnp.int32),
            pltpu.VMEM((EBUF,), jnp.float32),
            pltpu.SemaphoreType.DMA,
            pltpu.SemaphoreType.DMA,
        ],
    )(_gcn_edges_body)
    return f(sd, ea, g)


def _final_body(accp_ref, dinv_ref, sp_ref, bgcn_ref, out_ref):
    acc = jnp.sum(accp_ref[...], axis=0)
    res = jax.nn.sigmoid(dinv_ref[...] * acc + sp_ref[...]
                         + bgcn_ref[...][0, 0])
    out_ref[...] = res[:NN]


def _final(accp, dinv, sp, b_gcn):
    return pl.pallas_call(
        _final_body,
        out_shape=jax.ShapeDtypeStruct((NN,), jnp.float32),
    )(accp, dinv, sp, b_gcn)


def kernel(x, edge_index, edge_attr, batch, W_gat, att_src, att_dst, W_edge,
           att_edge, b_gat, prelu_a, W_gcn, b_gcn):
    ea2 = edge_attr.reshape(EE // 128, 128)
    sd = edge_index.reshape(2 * EE)
    xt, ad_arr, aself, consts = _prep(x, ea2, W_gat, att_src, att_dst,
                                      W_edge, att_edge)
    denomp, nump, degp = _gat_edges(sd, edge_attr, ad_arr, xt, consts)
    g, dinv, sp = _mid(degp, denomp, nump, aself, xt, consts,
                       W_gat, b_gat, prelu_a.reshape(1, 1), W_gcn)
    accp = _gcn_edges(sd, edge_attr, g)
    out = _final(accp, dinv, sp, b_gcn.reshape(1, 1))
    return out.reshape(NN, 1)


# speedup vs baseline: 138.5714x; 1.0165x over previous
import functools

import jax
import jax.numpy as jnp
from jax import lax
from jax.experimental import pallas as pl
from jax.experimental.pallas import tpu as pltpu
from jax.experimental.pallas import tpu_sc as plsc

NN = 10000
NNP = 10240
EE = 160000
HH = 512
NF = 4
NC = 2
NS = 16
NW = NC * NS
LN = 16
EPT = EE // NW
EBUF = 5008
EWIN = 5376
EIBUF = 5504
EILAST = EE - EWIN
CHF = EPT // LN
TAIL = EPT - CHF * LN
NB = 2048


def _prep_body(x_ref, ea2_ref, wg_ref, asrc_ref, adst_ref, wedge_ref,
               aedge_ref, xt_out, ad_out, aself_out, consts_out):
    x = x_ref[...]
    wg = wg_ref[...]
    vs = jnp.sum(wg * asrc_ref[...][None, :], axis=1)
    vd = jnp.sum(wg * adst_ref[...][None, :], axis=1)
    c_e = jnp.sum(wedge_ref[...][0, :] * aedge_ref[...])
    xt = lax.dot_general(jnp.eye(NF, dtype=jnp.float32), x,
                         (((1,), (1,)), ((), ())),
                         preferred_element_type=jnp.float32)
    xtp = jnp.concatenate(
        [xt, jnp.zeros((NF, NNP - NN), jnp.float32)], axis=1)
    a_s = (vs[0] * xtp[0] + vs[1] * xtp[1]
           + vs[2] * xtp[2] + vs[3] * xtp[3])
    a_d = (vd[0] * xtp[0] + vd[1] * xtp[1]
           + vd[2] * xtp[2] + vd[3] * xtp[3])
    ea2 = ea2_ref[...]
    mean_ea = jnp.sum(ea2) / EE
    asl = a_s + a_d + c_e * mean_ea
    asl = jnp.where(asl > 0, asl, 0.2 * asl)
    ub = jnp.max(a_s) + jnp.max(a_d) + jnp.max(c_e * ea2)
    gub = jnp.maximum(jnp.where(ub > 0, ub, 0.2 * ub), jnp.max(asl))
    xt_out[...] = xtp
    ad_out[...] = a_d
    aself_out[...] = asl
    rows = [jnp.full((128,), c_e, jnp.float32),
            jnp.full((128,), gub, jnp.float32),
            jnp.full((128,), vs[0], jnp.float32),
            jnp.full((128,), vs[1], jnp.float32),
            jnp.full((128,), vs[2], jnp.float32),
            jnp.full((128,), vs[3], jnp.float32),
            jnp.zeros((128,), jnp.float32),
            jnp.zeros((128,), jnp.float32)]
    consts_out[...] = jnp.stack(rows)


def _prep(x, ea2, w_gat, att_src, att_dst, w_edge, att_edge):
    return pl.pallas_call(
        _prep_body,
        out_shape=(jax.ShapeDtypeStruct((NF, NNP), jnp.float32),
                   jax.ShapeDtypeStruct((NNP,), jnp.float32),
                   jax.ShapeDtypeStruct((NNP,), jnp.float32),
                   jax.ShapeDtypeStruct((8, 128), jnp.float32)),
    )(x, ea2, w_gat, att_src, att_dst, w_edge, att_edge)


def _gat_edges_body(ei_hbm, ea_hbm, ad_hbm, xt_hbm, consts_hbm,
                    denomp, nump, degp,
                    ad_t, x0_t, x1_t, x2_t, x3_t,
                    den_a, n0_a, n1_a, n2_a, n3_a, deg_a,
                    sd_b, ea_b, consts_v, sem_t, sem_e):
    wid = lax.axis_index("c") * NS + lax.axis_index("s")
    base = wid * EPT

    base_al = pl.multiple_of(
        jnp.minimum((base // 128) * 128, EILAST), 128)
    off0 = base - base_al
    ecs = (pltpu.async_copy(ei_hbm.at[:, pl.ds(base_al, EWIN)],
                            sd_b.at[:, pl.ds(0, EWIN)], sem_e),
           pltpu.async_copy(ea_hbm.at[pl.ds(base, EPT)],
                            ea_b.at[pl.ds(0, EPT)], sem_e))
    tcs = (pltpu.async_copy(ad_hbm, ad_t, sem_t),
           pltpu.async_copy(xt_hbm.at[0], x0_t, sem_t),
           pltpu.async_copy(xt_hbm.at[1], x1_t, sem_t),
           pltpu.async_copy(xt_hbm.at[2], x2_t, sem_t),
           pltpu.async_copy(xt_hbm.at[3], x3_t, sem_t),
           pltpu.async_copy(consts_hbm, consts_v, sem_t))

    @plsc.parallel_loop(0, NNP // LN, unroll=8)
    def _(j):
        z = jnp.zeros((LN,), jnp.float32)
        for r in (den_a, n0_a, n1_a, n2_a, n3_a, deg_a):
            r[pl.ds(j * LN, LN)] = z

    for t in tcs + ecs:
        t.wait()
    c_e = consts_v[0, pl.ds(0, LN)]
    gub = consts_v[1, pl.ds(0, LN)]
    vs0 = consts_v[2, pl.ds(0, LN)]
    vs1 = consts_v[3, pl.ds(0, LN)]
    vs2 = consts_v[4, pl.ds(0, LN)]
    vs3 = consts_v[5, pl.ds(0, LN)]

    def chunk(i, mask):
        s = sd_b[0, pl.ds(off0 + i * LN, LN)]
        d = sd_b[1, pl.ds(off0 + i * LN, LN)]
        e = ea_b[pl.ds(i * LN, LN)]
        x0 = plsc.load_gather(x0_t, [s], mask=mask)
        x1 = plsc.load_gather(x1_t, [s], mask=mask)
        x2 = plsc.load_gather(x2_t, [s], mask=mask)
        x3 = plsc.load_gather(x3_t, [s], mask=mask)
        adv = plsc.load_gather(ad_t, [d], mask=mask)
        al = x0 * vs0 + x1 * vs1 + x2 * vs2 + x3 * vs3 + adv + c_e * e
        al = jnp.where(al > 0, al, 0.2 * al)
        ae = jnp.exp(al - gub)
        plsc.addupdate_scatter(den_a, [d], ae, mask=mask)
        plsc.addupdate_scatter(n0_a, [d], ae * x0, mask=mask)
        plsc.addupdate_scatter(n1_a, [d], ae * x1, mask=mask)
        plsc.addupdate_scatter(n2_a, [d], ae * x2, mask=mask)
        plsc.addupdate_scatter(n3_a, [d], ae * x3, mask=mask)
        plsc.addupdate_scatter(deg_a, [d], e, mask=mask)

    @plsc.parallel_loop(0, CHF, unroll=4)
    def _(i):
        chunk(i, None)

    chunk(CHF, lax.iota(jnp.int32, LN) < TAIL)

    pltpu.sync_copy(den_a, denomp.at[wid])
    pltpu.sync_copy(n0_a, nump.at[0, wid])
    pltpu.sync_copy(n1_a, nump.at[1, wid])
    pltpu.sync_copy(n2_a, nump.at[2, wid])
    pltpu.sync_copy(n3_a, nump.at[3, wid])
    pltpu.sync_copy(deg_a, degp.at[wid])


def _gat_edges(ei, ea, ad_arr, xt, consts):
    f = functools.partial(
        pl.kernel,
        out_type=(jax.ShapeDtypeStruct((NW, NNP), jnp.float32),
                  jax.ShapeDtypeStruct((NF, NW, NNP), jnp.float32),
                  jax.ShapeDtypeStruct((NW, NNP), jnp.float32)),
        mesh=plsc.VectorSubcoreMesh(core_axis_name="c", subcore_axis_name="s",
                                    num_cores=NC, num_subcores=NS),
        compiler_params=pltpu.CompilerParams(needs_layout_passes=False),
        scratch_types=[
            pltpu.VMEM((NNP,), jnp.float32),
            pltpu.VMEM((NNP,), jnp.float32),
            pltpu.VMEM((NNP,), jnp.float32),
            pltpu.VMEM((NNP,), jnp.float32),
            pltpu.VMEM((NNP,), jnp.float32),
            pltpu.VMEM((NNP,), jnp.float32),
            pltpu.VMEM((NNP,), jnp.float32),
            pltpu.VMEM((NNP,), jnp.float32),
            pltpu.VMEM((NNP,), jnp.float32),
            pltpu.VMEM((NNP,), jnp.float32),
            pltpu.VMEM((NNP,), jnp.float32),
            pltpu.VMEM((2, EIBUF), jnp.int32),
            pltpu.VMEM((EBUF,), jnp.float32),
            pltpu.VMEM((8, 128), jnp.float32),
            pltpu.SemaphoreType.DMA,
            pltpu.SemaphoreType.DMA,
        ],
    )(_gat_edges_body)
    return f(ei, ea, ad_arr, xt, consts)


def _mid_body(degp_ref, denomp_ref, nump_ref, aself_ref, xt_ref, consts_ref,
              wg_ref, bg_ref, pa_ref, wgcn_ref, g_out, dinv_out, sp_out):
    deg = jnp.sum(degp_ref[...], axis=0) + 1.0
    dinv = lax.rsqrt(deg)
    gub = consts_ref[...][1, 0]
    es = jnp.exp(aself_ref[...] - gub)
    denom = jnp.sum(denomp_ref[...], axis=0) + es
    numt = jnp.sum(nump_ref[...], axis=1) + es[None, :] * xt_ref[...]
    acc4 = numt / (denom + 1e-16)[None, :]
    x1 = lax.dot_general(acc4, wg_ref[...], (((0,), (0,)), ((), ())),
                         preferred_element_type=jnp.float32)
    x1 = x1 + bg_ref[...][None, :]
    pa = pa_ref[...][0, 0]
    x2 = jnp.where(x1 > 0, x1, pa * x1)
    h2 = jnp.sum(x2 * wgcn_ref[...][:, 0][None, :], axis=1)
    g_out[...] = dinv * h2
    dinv_out[...] = dinv
    sp_out[...] = dinv * dinv * h2


def _mid(degp, denomp, nump, aself, xt, consts, w_gat, b_gat, pa, w_gcn):
    grid = NNP // NB
    return pl.pallas_call(
        _mid_body,
        grid=(grid,),
        in_specs=[
            pl.BlockSpec((NW, NB), lambda i: (0, i)),
            pl.BlockSpec((NW, NB), lambda i: (0, i)),
            pl.BlockSpec((NF, NW, NB), lambda i: (0, 0, i)),
            pl.BlockSpec((NB,), lambda i: (i,)),
            pl.BlockSpec((NF, NB), lambda i: (0, i)),
            pl.BlockSpec((8, 128), lambda i: (0, 0)),
            pl.BlockSpec((NF, HH), lambda i: (0, 0)),
            pl.BlockSpec((HH,), lambda i: (0,)),
            pl.BlockSpec((1, 1), lambda i: (0, 0)),
            pl.BlockSpec((HH, 1), lambda i: (0, 0)),
        ],
        out_specs=[
            pl.BlockSpec((NB,), lambda i: (i,)),
            pl.BlockSpec((NB,), lambda i: (i,)),
            pl.BlockSpec((NB,), lambda i: (i,)),
        ],
        out_shape=(jax.ShapeDtypeStruct((NNP,), jnp.float32),
                   jax.ShapeDtypeStruct((NNP,), jnp.float32),
                   jax.ShapeDtypeStruct((NNP,), jnp.float32)),
    )(degp, denomp, nump, aself, xt, consts, w_gat, b_gat, pa, w_gcn)


def _gcn_edges_body(ei_hbm, ea_hbm, g_hbm, accp,
                    g_t, acc_a, sd_b, ea_b, sem_t, sem_e):
    wid = lax.axis_index("c") * NS + lax.axis_index("s")
    base = wid * EPT

    base_al = pl.multiple_of(
        jnp.minimum((base // 128) * 128, EILAST), 128)
    off0 = base - base_al
    ecs = (pltpu.async_copy(ei_hbm.at[:, pl.ds(base_al, EWIN)],
                            sd_b.at[:, pl.ds(0, EWIN)], sem_e),
           pltpu.async_copy(ea_hbm.at[pl.ds(base, EPT)],
                            ea_b.at[pl.ds(0, EPT)], sem_e))
    tc = pltpu.async_copy(g_hbm, g_t, sem_t)

    @plsc.parallel_loop(0, NNP // LN, unroll=8)
    def _(j):
        acc_a[pl.ds(j * LN, LN)] = jnp.zeros((LN,), jnp.float32)

    tc.wait()
    for dsc in ecs:
        dsc.wait()

    def chunk(i, mask):
        s = sd_b[0, pl.ds(off0 + i * LN, LN)]
        d = sd_b[1, pl.ds(off0 + i * LN, LN)]
        e = ea_b[pl.ds(i * LN, LN)]
        gv = plsc.load_gather(g_t, [s], mask=mask)
        plsc.addupdate_scatter(acc_a, [d], e * gv, mask=mask)

    @plsc.parallel_loop(0, CHF, unroll=8)
    def _(i):
        chunk(i, None)

    chunk(CHF, lax.iota(jnp.int32, LN) < TAIL)

    pltpu.sync_copy(acc_a, accp.at[wid])


def _gcn_edges(ei, ea, g):
    f = functools.partial(
        pl.kernel,
        out_type=jax.ShapeDtypeStruct((NW, NNP), jnp.float32),
        mesh=plsc.VectorSubcoreMesh(core_axis_name="c", subcore_axis_name="s",
                                    num_cores=NC, num_subcores=NS),
        compiler_params=pltpu.CompilerParams(needs_layout_passes=False),
        scratch_types=[
            pltpu.VMEM((NNP,), jnp.float32),
            pltpu.VMEM((NNP,), jnp.float32),
            pltpu.VMEM((2, EIBUF), jnp.int32),
            pltpu.VMEM((EBUF,), jnp.float32),
            pltpu.SemaphoreType.DMA,
            pltpu.SemaphoreType.DMA,
        ],
    )(_gcn_edges_body)
    return f(ei, ea, g)


def _final_body(accp_ref, dinv_ref, sp_ref, bgcn_ref, out_ref):
    acc = jnp.sum(accp_ref[...], axis=0)
    res = jax.nn.sigmoid(dinv_ref[...] * acc + sp_ref[...]
                         + bgcn_ref[...][0, 0])
    out_ref[...] = res[:NN]


def _final(accp, dinv, sp, b_gcn):
    return pl.pallas_call(
        _final_body,
        out_shape=jax.ShapeDtypeStruct((NN,), jnp.float32),
    )(accp, dinv, sp, b_gcn)


def kernel(x, edge_index, edge_attr, batch, W_gat, att_src, att_dst, W_edge,
           att_edge, b_gat, prelu_a, W_gcn, b_gcn):
    ea2 = edge_attr.reshape(EE // 128, 128)
    xt, ad_arr, aself, consts = _prep(x, ea2, W_gat, att_src, att_dst,
                                      W_edge, att_edge)
    denomp, nump, degp = _gat_edges(edge_index, edge_attr, ad_arr, xt, consts)
    g, dinv, sp = _mid(degp, denomp, nump, aself, xt, consts,
                       W_gat, b_gat, prelu_a.reshape(1, 1), W_gcn)
    accp = _gcn_edges(edge_index, edge_attr, g)
    out = _final(accp, dinv, sp, b_gcn.reshape(1, 1))
    return out.reshape(NN, 1)
